# baseline (speedup 1.0000x reference)
import jax
import jax.numpy as jnp
import numpy as np
from jax import lax
from jax.experimental import pallas as pl
from jax.experimental.pallas import tpu as pltpu
from jax.experimental.pallas import tpu_sc as plsc

NUM_NUM = 13
NUM_CAT = 26
VOCAB = 100000
EMB = 32
BATCH = 16384
SLOTS = NUM_CAT + 1

NC = 2
NS = 16
NW = NC * NS
ROWS_PER_W = BATCH // NW
LOOKUPS_PER_W = ROWS_PER_W * NUM_CAT
CHUNK = 128
NCHUNK = LOOKUPS_PER_W // CHUNK
MLP_NCHUNK = ROWS_PER_W // CHUNK


def _mlp_body(x_ref, w1_ref, b1_ref, w2_ref, b2_ref, w3_ref, b3_ref, o_ref):
    h = jnp.maximum(
        jnp.dot(x_ref[...], w1_ref[...], preferred_element_type=jnp.float32)
        + b1_ref[...], 0.0)
    h = jnp.maximum(
        jnp.dot(h, w2_ref[...], preferred_element_type=jnp.float32)
        + b2_ref[...], 0.0)
    o_ref[...] = jnp.maximum(
        jnp.dot(h, w3_ref[...], preferred_element_type=jnp.float32)
        + b3_ref[...], 0.0)


def _mlp(numerical_input, W1, b1, W2, b2, W3, b3):
    tb = 2048
    grid = (BATCH // tb,)
    return pl.pallas_call(
        _mlp_body,
        grid=grid,
        in_specs=[
            pl.BlockSpec((tb, NUM_NUM), lambda i: (i, 0)),
            pl.BlockSpec((NUM_NUM, 512), lambda i: (0, 0)),
            pl.BlockSpec((1, 512), lambda i: (0, 0)),
            pl.BlockSpec((512, 256), lambda i: (0, 0)),
            pl.BlockSpec((1, 256), lambda i: (0, 0)),
            pl.BlockSpec((256, EMB), lambda i: (0, 0)),
            pl.BlockSpec((1, EMB), lambda i: (0, 0)),
        ],
        out_specs=pl.BlockSpec((tb, EMB), lambda i: (i, 0)),
        out_shape=jax.ShapeDtypeStruct((BATCH, EMB), jnp.float32),
    )(numerical_input, W1, b1.reshape(1, -1), W2, b2.reshape(1, -1),
      W3, b3.reshape(1, -1))


DEPTH = 8
LAG = 4


def _sc_body(table, gidx, oidx, sidx, mlp, out,
             gidx_v, oidx_v, sidx_v, gbuf, mbuf, gsem, ssem):
    wid = lax.axis_index("s") * NC + lax.axis_index("c")
    pltpu.sync_copy(gidx.at[wid], gidx_v)
    pltpu.sync_copy(oidx.at[wid], oidx_v)
    pltpu.sync_copy(sidx.at[wid], sidx_v)
    pltpu.sync_copy(mlp.at[pl.ds(wid * ROWS_PER_W, ROWS_PER_W)], mbuf)

    for k in range(DEPTH):
        pltpu.async_copy(table.at[gidx_v.at[k]], gbuf.at[k], gsem.at[k])

    def step(j, carry):
        slot = j % DEPTH
        pltpu.make_async_copy(table.at[gidx_v.at[j]], gbuf.at[slot],
                              gsem.at[slot]).wait()
        pltpu.async_copy(gbuf.at[slot], out.at[oidx_v.at[j]], ssem.at[slot])
        jj = j - LAG

        @pl.when(jj >= 0)
        def _():
            sl = jj % DEPTH
            pltpu.make_async_copy(gbuf.at[sl], out.at[oidx_v.at[jj]],
                                  ssem.at[sl]).wait()

            @pl.when(jj + DEPTH < NCHUNK)
            def _():
                pltpu.async_copy(table.at[gidx_v.at[jj + DEPTH]], gbuf.at[sl],
                                 gsem.at[sl])
        return carry

    lax.fori_loop(0, NCHUNK, step, 0)

    def drain(j, carry):
        sl = j % DEPTH
        pltpu.make_async_copy(gbuf.at[sl], out.at[oidx_v.at[j]],
                              ssem.at[sl]).wait()
        return carry

    lax.fori_loop(NCHUNK - LAG, NCHUNK, drain, 0)

    def mstep(j, carry):
        pltpu.async_copy(mbuf.at[pl.ds(j * CHUNK, CHUNK)],
                         out.at[sidx_v.at[j]], ssem.at[0]).wait()
        return carry

    lax.fori_loop(0, MLP_NCHUNK, mstep, 0)


def _sc_assemble(table_flat, gidx, oidx, sidx, mlp):
    mesh = plsc.VectorSubcoreMesh(core_axis_name="c", subcore_axis_name="s")
    run = pl.kernel(
        _sc_body,
        mesh=mesh,
        compiler_params=pltpu.CompilerParams(use_tc_tiling_on_sc=False),
        out_type=jax.ShapeDtypeStruct((BATCH * SLOTS, EMB), jnp.float32),
        scratch_types=[
            pltpu.VMEM((NCHUNK, CHUNK), jnp.int32),
            pltpu.VMEM((NCHUNK, CHUNK), jnp.int32),
            pltpu.VMEM((MLP_NCHUNK, CHUNK), jnp.int32),
            pltpu.VMEM((DEPTH, CHUNK, EMB), jnp.float32),
            pltpu.VMEM((ROWS_PER_W, EMB), jnp.float32),
            pltpu.SemaphoreType.DMA((DEPTH,)),
            pltpu.SemaphoreType.DMA((DEPTH,)),
        ],
    )
    return run(table_flat, gidx, oidx, sidx, mlp)


def _static_indices():
    b = np.arange(BATCH, dtype=np.int32)
    t = np.arange(NUM_CAT, dtype=np.int32)
    oidx = (b[:, None] * SLOTS + t[None, :] + 1).reshape(NW, NCHUNK, CHUNK)
    sidx = (b * SLOTS).reshape(NW, MLP_NCHUNK, CHUNK)
    return oidx, sidx

_OIDX, _SIDX = _static_indices()


def kernel(numerical_input, categorical_inputs, tables, W1, b1, W2, b2, W3, b3):
    mlp = _mlp(numerical_input, W1, b1, W2, b2, W3, b3)
    cat = categorical_inputs.astype(jnp.int32)
    offs = (np.arange(NUM_CAT, dtype=np.int32) * VOCAB)[None, :]
    gidx = (cat + offs).reshape(NW, NCHUNK, CHUNK)
    out_flat = _sc_assemble(
        tables.reshape(NUM_CAT * VOCAB, EMB), gidx,
        jnp.asarray(_OIDX), jnp.asarray(_SIDX), mlp)
    bottom_output = out_flat.reshape(BATCH, SLOTS, EMB)
    return (bottom_output, mlp)


# speedup vs baseline: 3.3709x; 3.3709x over previous
import jax
import jax.numpy as jnp
from jax import lax
from jax.experimental import pallas as pl
from jax.experimental.pallas import tpu as pltpu
from jax.experimental.pallas import tpu_sc as plsc

NUM_NUM = 13
NUM_CAT = 26
VOCAB = 100000
EMB = 32
BATCH = 16384
SLOTS = NUM_CAT + 1

NC = 2
NS = 16
NW = NC * NS
NPAIRS = NUM_CAT * EMB
PAIRS_PER_W = NPAIRS // NW
CHUNK = 2048
NCHUNK = BATCH // CHUNK
LANES = 16


def _mlp_body(x_ref, w1_ref, b1_ref, w2_ref, b2_ref, w3_ref, b3_ref, o_ref):
    tdot = lambda w, x: lax.dot_general(
        w, x, (((0,), (0,)), ((), ())), preferred_element_type=jnp.float32)
    h = jnp.maximum(tdot(w1_ref[...], x_ref[...]) + b1_ref[...], 0.0)
    h = jnp.maximum(tdot(w2_ref[...], h) + b2_ref[...], 0.0)
    o_ref[...] = jnp.maximum(tdot(w3_ref[...], h) + b3_ref[...], 0.0)


def _mlp_t(x_t, W1, b1, W2, b2, W3, b3):
    tb = 2048
    grid = (BATCH // tb,)
    return pl.pallas_call(
        _mlp_body,
        grid=grid,
        in_specs=[
            pl.BlockSpec((NUM_NUM, tb), lambda i: (0, i)),
            pl.BlockSpec((NUM_NUM, 512), lambda i: (0, 0)),
            pl.BlockSpec((512, 1), lambda i: (0, 0)),
            pl.BlockSpec((512, 256), lambda i: (0, 0)),
            pl.BlockSpec((256, 1), lambda i: (0, 0)),
            pl.BlockSpec((256, EMB), lambda i: (0, 0)),
            pl.BlockSpec((EMB, 1), lambda i: (0, 0)),
        ],
        out_specs=pl.BlockSpec((EMB, tb), lambda i: (0, i)),
        out_shape=jax.ShapeDtypeStruct((EMB, BATCH), jnp.float32),
    )(x_t, W1, b1.reshape(-1, 1), W2, b2.reshape(-1, 1),
      W3, b3.reshape(-1, 1))


def _sc_body(tables_t, cat_t, mlp_t, out, vocab_v, idx_v, oval_v):
    wid = lax.axis_index("s") * NC + lax.axis_index("c")
    p0 = wid * PAIRS_PER_W

    def pair_body(i, prev_t):
        p = p0 + i
        t = p // EMB
        e = p % EMB
        pltpu.sync_copy(tables_t.at[t, e], vocab_v)

        @pl.when(t != prev_t)
        def _():
            pltpu.sync_copy(cat_t.at[t], idx_v)

        def chunk_body(c, carry):
            def lane_body(k, carry2):
                iv = idx_v[pl.ds(c * CHUNK + k * LANES, LANES)]
                oval_v[pl.ds(k * LANES, LANES)] = plsc.load_gather(
                    vocab_v, [iv])
                return carry2

            lax.fori_loop(0, CHUNK // LANES, lane_body, 0, unroll=8)
            pltpu.sync_copy(oval_v, out.at[t + 1, e, pl.ds(c * CHUNK, CHUNK)])
            return carry

        lax.fori_loop(0, NCHUNK, chunk_body, 0)
        return t

    lax.fori_loop(0, PAIRS_PER_W, pair_body, jnp.int32(-1))

    def mlp_chunk(c, carry):
        pltpu.sync_copy(mlp_t.at[wid, pl.ds(c * CHUNK, CHUNK)], oval_v)
        pltpu.sync_copy(oval_v, out.at[0, wid, pl.ds(c * CHUNK, CHUNK)])
        return carry

    lax.fori_loop(0, NCHUNK, mlp_chunk, 0)


def _sc_assemble(tables_t, cat_t, mlp_t):
    mesh = plsc.VectorSubcoreMesh(core_axis_name="c", subcore_axis_name="s")
    run = pl.kernel(
        _sc_body,
        mesh=mesh,
        compiler_params=pltpu.CompilerParams(use_tc_tiling_on_sc=True,
                                             needs_layout_passes=False),
        out_type=jax.ShapeDtypeStruct((SLOTS, EMB, BATCH), jnp.float32),
        scratch_types=[
            pltpu.VMEM((VOCAB,), jnp.float32),
            pltpu.VMEM((BATCH,), jnp.int32),
            pltpu.VMEM((CHUNK,), jnp.float32),
        ],
    )
    return run(tables_t, cat_t, mlp_t)


def kernel(numerical_input, categorical_inputs, tables, W1, b1, W2, b2, W3, b3):
    cat_t = categorical_inputs.astype(jnp.int32).T
    tables_t = jnp.transpose(tables, (0, 2, 1))
    mlp_t = _mlp_t(numerical_input.T, W1, b1, W2, b2, W3, b3)
    out_t = _sc_assemble(tables_t, cat_t, mlp_t)
    bottom_output = jnp.transpose(out_t, (2, 0, 1))
    return (bottom_output, mlp_t.T)


# speedup vs baseline: 6.1702x; 1.8305x over previous
import jax
import jax.numpy as jnp
from jax import lax
from jax.experimental import pallas as pl
from jax.experimental.pallas import tpu as pltpu
from jax.experimental.pallas import tpu_sc as plsc

NUM_NUM = 13
NUM_CAT = 26
VOCAB = 100000
EMB = 32
BATCH = 16384
SLOTS = NUM_CAT + 1

NC = 2
NS = 16
NW = NC * NS
NPAIRS = NUM_CAT * EMB
PAIRS_PER_W = NPAIRS // NW
CHUNK = 2048
NCHUNK = BATCH // CHUNK
LANES = 16


def _mlp_body(x_ref, w1_ref, b1_ref, w2_ref, b2_ref, w3_ref, b3_ref, o_ref):
    tdot = lambda w, x: lax.dot_general(
        w, x, (((0,), (0,)), ((), ())), preferred_element_type=jnp.float32)
    h = jnp.maximum(tdot(w1_ref[...], x_ref[...]) + b1_ref[...], 0.0)
    h = jnp.maximum(tdot(w2_ref[...], h) + b2_ref[...], 0.0)
    o_ref[...] = jnp.maximum(tdot(w3_ref[...], h) + b3_ref[...], 0.0)


def _mlp_t(x_t, W1, b1, W2, b2, W3, b3):
    tb = 2048
    grid = (BATCH // tb,)
    return pl.pallas_call(
        _mlp_body,
        grid=grid,
        in_specs=[
            pl.BlockSpec((NUM_NUM, tb), lambda i: (0, i)),
            pl.BlockSpec((NUM_NUM, 512), lambda i: (0, 0)),
            pl.BlockSpec((512, 1), lambda i: (0, 0)),
            pl.BlockSpec((512, 256), lambda i: (0, 0)),
            pl.BlockSpec((256, 1), lambda i: (0, 0)),
            pl.BlockSpec((256, EMB), lambda i: (0, 0)),
            pl.BlockSpec((EMB, 1), lambda i: (0, 0)),
        ],
        out_specs=pl.BlockSpec((EMB, tb), lambda i: (0, i)),
        out_shape=jax.ShapeDtypeStruct((EMB, BATCH), jnp.float32),
    )(x_t, W1, b1.reshape(-1, 1), W2, b2.reshape(-1, 1),
      W3, b3.reshape(-1, 1))


def _sc_body(tables_t, cat_t, mlp_t, out, vocab_v, idx_v, oval_v):
    wid = lax.axis_index("s") * NC + lax.axis_index("c")
    p0 = wid * PAIRS_PER_W

    def pair_body(i, prev_t):
        p = p0 + i
        t = p // EMB
        e = p % EMB
        pltpu.sync_copy(tables_t.at[t, e], vocab_v)

        @pl.when(t != prev_t)
        def _():
            pltpu.sync_copy(cat_t.at[t], idx_v)

        def chunk_body(c, carry):
            @plsc.parallel_loop(0, CHUNK // LANES, unroll=8)
            def _(k):
                iv = idx_v[pl.ds(c * CHUNK + k * LANES, LANES)]
                oval_v[pl.ds(k * LANES, LANES)] = plsc.load_gather(
                    vocab_v, [iv])
            pltpu.sync_copy(oval_v, out.at[t + 1, e, pl.ds(c * CHUNK, CHUNK)])
            return carry

        lax.fori_loop(0, NCHUNK, chunk_body, 0)
        return t

    lax.fori_loop(0, PAIRS_PER_W, pair_body, jnp.int32(-1))

    def mlp_chunk(c, carry):
        pltpu.sync_copy(mlp_t.at[wid, pl.ds(c * CHUNK, CHUNK)], oval_v)
        pltpu.sync_copy(oval_v, out.at[0, wid, pl.ds(c * CHUNK, CHUNK)])
        return carry

    lax.fori_loop(0, NCHUNK, mlp_chunk, 0)


def _sc_assemble(tables_t, cat_t, mlp_t):
    mesh = plsc.VectorSubcoreMesh(core_axis_name="c", subcore_axis_name="s")
    run = pl.kernel(
        _sc_body,
        mesh=mesh,
        compiler_params=pltpu.CompilerParams(use_tc_tiling_on_sc=True,
                                             needs_layout_passes=False),
        out_type=jax.ShapeDtypeStruct((SLOTS, EMB, BATCH), jnp.float32),
        scratch_types=[
            pltpu.VMEM((VOCAB,), jnp.float32),
            pltpu.VMEM((BATCH,), jnp.int32),
            pltpu.VMEM((CHUNK,), jnp.float32),
        ],
    )
    return run(tables_t, cat_t, mlp_t)


def kernel(numerical_input, categorical_inputs, tables, W1, b1, W2, b2, W3, b3):
    cat_t = categorical_inputs.astype(jnp.int32).T
    tables_t = jnp.transpose(tables, (0, 2, 1))
    mlp_t = _mlp_t(numerical_input.T, W1, b1, W2, b2, W3, b3)
    out_t = _sc_assemble(tables_t, cat_t, mlp_t)
    bottom_output = jnp.transpose(out_t, (2, 0, 1))
    return (bottom_output, mlp_t.T)
